# baseline (speedup 1.0000x reference)
import functools

import jax
import jax.numpy as jnp
from jax.experimental import pallas as pl

B, C, H, W = 32, 384, 56, 56
E = 16
K = 2
ROWS = B * C
BLK = 256


def _stats_kernel(x_ref, out_ref):
    x = x_ref[...]
    z_row = jnp.zeros((BLK, 1, W), dtype=x.dtype)
    z_col = jnp.zeros((BLK, H, 1), dtype=x.dtype)
    up = jnp.concatenate([z_row, x[:, :-1, :]], axis=1)
    down = jnp.concatenate([x[:, 1:, :], z_row], axis=1)
    left = jnp.concatenate([z_col, x[:, :, :-1]], axis=2)
    right = jnp.concatenate([x[:, :, 1:], z_col], axis=2)
    lap = 4.0 * x - up - down - left - right
    inv = jnp.float32(1.0 / (H * W))
    s_mean = jnp.sum(x, axis=(1, 2)) * inv
    s_freq = jnp.sum(jnp.abs(lap), axis=(1, 2)) * inv
    out_ref[...] = jnp.stack([s_mean, s_freq], axis=1)


def _router_kernel(c_ref, w1_ref, b1_ref, w2_ref, b2_ref, rw_ref, idx_ref):
    combined = c_ref[...]
    h1 = jax.lax.dot_general(
        combined, w1_ref[...],
        (((1,), (1,)), ((), ())),
        preferred_element_type=jnp.float32,
    ) + b1_ref[...]
    h1 = jnp.maximum(h1, 0.0)
    logits = jax.lax.dot_general(
        h1, w2_ref[...],
        (((1,), (1,)), ((), ())),
        preferred_element_type=jnp.float32,
    ) + b2_ref[...]

    iota = jax.lax.broadcasted_iota(jnp.int32, (B, E), 1)
    m1 = jnp.max(logits, axis=1, keepdims=True)
    i1 = jnp.min(jnp.where(logits == m1, iota, E), axis=1, keepdims=True)
    neg = jnp.float32(-3.0e38)
    masked = jnp.where(iota == i1, neg, logits)
    m2 = jnp.max(masked, axis=1, keepdims=True)
    i2 = jnp.min(jnp.where(masked == m2, iota, E), axis=1, keepdims=True)

    e2 = jnp.exp(m2 - m1)
    denom = 1.0 + e2
    p1 = 1.0 / denom
    p2 = e2 / denom

    rw_ref[...] = jnp.where(iota == i1, p1, 0.0) + jnp.where(iota == i2, p2, 0.0)
    idx_ref[...] = jnp.concatenate([i1, i2], axis=1)


@functools.partial(jax.jit, static_argnames=("interpret",))
def _run(x, W1, b1, W2, b2, interpret=False):
    xr = x.reshape(ROWS, H, W)
    stats = pl.pallas_call(
        _stats_kernel,
        grid=(ROWS // BLK,),
        in_specs=[pl.BlockSpec((BLK, H, W), lambda i: (i, 0, 0))],
        out_specs=pl.BlockSpec((BLK, 2), lambda i: (i, 0)),
        out_shape=jax.ShapeDtypeStruct((ROWS, 2), jnp.float32),
        interpret=interpret,
    )(xr)
    stats = stats.reshape(B, C, 2)
    combined = jnp.concatenate([stats[:, :, 0], stats[:, :, 1]], axis=1)

    rw, idx = pl.pallas_call(
        _router_kernel,
        in_specs=[
            pl.BlockSpec((B, 2 * C), lambda: (0, 0)),
            pl.BlockSpec((C, 2 * C), lambda: (0, 0)),
            pl.BlockSpec((1, C), lambda: (0, 0)),
            pl.BlockSpec((E, C), lambda: (0, 0)),
            pl.BlockSpec((1, E), lambda: (0, 0)),
        ],
        out_specs=[
            pl.BlockSpec((B, E), lambda: (0, 0)),
            pl.BlockSpec((B, K), lambda: (0, 0)),
        ],
        out_shape=[
            jax.ShapeDtypeStruct((B, E), jnp.float32),
            jax.ShapeDtypeStruct((B, K), jnp.int32),
        ],
        interpret=interpret,
    )(combined, W1, b1.reshape(1, C), W2, b2.reshape(1, E))
    return rw.reshape(B, E, 1, 1), idx.reshape(B, K, 1, 1)


def kernel(x, W1, b1, W2, b2):
    return _run(x, W1, b1, W2, b2)
